# baseline (speedup 1.0000x reference)
import functools

import jax
import jax.numpy as jnp
from jax import lax
from jax.experimental import pallas as pl
from jax.experimental.pallas import tpu as pltpu
from jax.experimental.pallas import tpu_sc as plsc

B = 16384
D = 256
CB = 3
NC = 2
NS = 16
NW = NC * NS
BPW = B // NW
CHUNK = 64
NCH = BPW // CHUNK
LANES = 16
NBUF = 6
LOOKAHEAD = 3
WR = 128
MCW = 8192 * D // WR
NSTEP = NCH


def _body(item_hbm, cl, w, out_hbm, item_v, codes_v, idx2_v, embs_and_sems):
    embufs = embs_and_sems[:NBUF]
    sem_c = embs_and_sems[NBUF]
    gsems = embs_and_sems[NBUF + 1:2 * NBUF + 1]
    osems = embs_and_sems[2 * NBUF + 1:]
    wid = lax.axis_index("s") * NC + lax.axis_index("c")
    evens = 2 * lax.iota(jnp.int32, LANES)

    pltpu.async_copy(item_hbm.at[pl.ds(wid * BPW, BPW)], item_v, sem_c).wait()

    cdescs = []
    for c in range(NCH):
        cdescs.append(pltpu.async_copy(
            cl.at[item_v.at[pl.ds(c * CHUNK, CHUNK)]],
            codes_v.at[c], sem_c))
    for d in cdescs:
        d.wait()

    for c in range(NCH):
        for g in range(CHUNK // LANES):
            cv = codes_v[c, pl.ds(g * LANES, LANES)]
            c2 = cv + cv
            plsc.store_scatter(idx2_v.at[c], [evens + 2 * g * LANES], c2)
            plsc.store_scatter(idx2_v.at[c],
                               [evens + (2 * g * LANES + 1)], c2 + 1)

    def fire(s):
        b = s % NBUF
        return pltpu.async_copy(w.at[idx2_v.at[s]], embufs[b], gsems[b])

    def fire_out(s):
        b = s % NBUF
        row0 = 2 * (wid * BPW + s * CHUNK)
        return pltpu.async_copy(
            embufs[b], out_hbm.at[pl.ds(row0, 2 * CHUNK)], osems[b])

    descs = {}
    odescs = {}
    for s in range(LOOKAHEAD):
        descs[s] = fire(s)
    for s in range(NSTEP):
        descs[s].wait()
        odescs[s] = fire_out(s)
        t = s + LOOKAHEAD
        if t < NSTEP:
            if t >= NBUF:
                odescs[t - NBUF].wait()
            descs[t] = fire(t)
    for s in range(max(0, NSTEP - NBUF), NSTEP):
        odescs[s].wait()


@jax.jit
def _rpq1(item, cl, w):
    mesh = plsc.VectorSubcoreMesh(
        core_axis_name="c", subcore_axis_name="s",
        num_cores=NC, num_subcores=NS)
    return pl.kernel(
        _body,
        out_type=jax.ShapeDtypeStruct((2 * B, WR), jnp.float32),
        mesh=mesh,
        scratch_types=[
            pltpu.VMEM((BPW,), jnp.int32),
            pltpu.VMEM((NCH, CHUNK), jnp.int32),
            pltpu.VMEM((NCH, 2 * CHUNK), jnp.int32),
            [pltpu.VMEM((2 * CHUNK, WR), jnp.float32)] * NBUF
            + [pltpu.SemaphoreType.DMA] * (1 + 2 * NBUF),
        ],
        compiler_params=pltpu.CompilerParams(needs_layout_passes=False),
    )(item, cl, w)


def kernel(item, code_list_0, code_list_1, code_list_2, W0, W1, W2):
    outs = []
    for cl, W in ((code_list_0, W0), (code_list_1, W1), (code_list_2, W2)):
        raw = _rpq1(item, cl, W.reshape(MCW, WR))
        outs.append(raw.reshape(B, D))
    return jnp.stack(outs, axis=-1)


# speedup vs baseline: 1.0996x; 1.0996x over previous
import functools

import jax
import jax.numpy as jnp
from jax import lax
from jax.experimental import pallas as pl
from jax.experimental.pallas import tpu as pltpu
from jax.experimental.pallas import tpu_sc as plsc

B = 16384
D = 256
CB = 3
NC = 2
NS = 16
NW = NC * NS
BPW = B // NW
CHUNK = 64
NCH = BPW // CHUNK
LANES = 16
NBUF = 7
LOOKAHEAD = 4
WR = 128
MCW = 8192 * D // WR
NSTEP = NCH * CB


def _body(item_hbm, cl0, cl1, cl2, w0, w1, w2, out_hbm,
          item_v, codes_v, idx2_v, embs_and_sems):
    embufs = embs_and_sems[:NBUF]
    sem_c = embs_and_sems[NBUF]
    gsems = embs_and_sems[NBUF + 1:2 * NBUF + 1]
    osems = embs_and_sems[2 * NBUF + 1:]
    wid = lax.axis_index("s") * NC + lax.axis_index("c")
    cls = (cl0, cl1, cl2)
    ws = (w0, w1, w2)
    evens = 2 * lax.iota(jnp.int32, LANES)

    pltpu.async_copy(item_hbm.at[pl.ds(wid * BPW, BPW)], item_v, sem_c).wait()

    cdescs = []
    for c in range(NCH):
        for i in range(CB):
            cdescs.append(pltpu.async_copy(
                cls[i].at[item_v.at[pl.ds(c * CHUNK, CHUNK)]],
                codes_v.at[i, c], sem_c))
    for d in cdescs:
        d.wait()

    for i in range(CB):
        for c in range(NCH):
            for g in range(CHUNK // LANES):
                cv = codes_v[i, c, pl.ds(g * LANES, LANES)]
                c2 = cv + cv
                plsc.store_scatter(idx2_v.at[i, c], [evens + 2 * g * LANES], c2)
                plsc.store_scatter(idx2_v.at[i, c],
                                   [evens + (2 * g * LANES + 1)], c2 + 1)

    steps = [(c, i) for c in range(NCH) for i in range(CB)]

    def fire(s):
        c, i = steps[s]
        b = s % NBUF
        return pltpu.async_copy(ws[i].at[idx2_v.at[i, c]], embufs[b], gsems[b])

    def fire_out(s):
        c, i = steps[s]
        b = s % NBUF
        row0 = i * 2 * B + 2 * (wid * BPW + c * CHUNK)
        return pltpu.async_copy(
            embufs[b], out_hbm.at[pl.ds(row0, 2 * CHUNK)], osems[b])

    descs = {}
    odescs = {}
    for s in range(LOOKAHEAD):
        descs[s] = fire(s)
    for s in range(NSTEP):
        descs[s].wait()
        odescs[s] = fire_out(s)
        t = s + LOOKAHEAD
        if t < NSTEP:
            if t >= NBUF:
                odescs[t - NBUF].wait()
            descs[t] = fire(t)
    for s in range(NSTEP - NBUF, NSTEP):
        odescs[s].wait()


@jax.jit
def _rpq(item, cl0, cl1, cl2, w0, w1, w2):
    mesh = plsc.VectorSubcoreMesh(
        core_axis_name="c", subcore_axis_name="s",
        num_cores=NC, num_subcores=NS)
    return pl.kernel(
        _body,
        out_type=jax.ShapeDtypeStruct((CB * 2 * B, WR), jnp.float32),
        mesh=mesh,
        scratch_types=[
            pltpu.VMEM((BPW,), jnp.int32),
            pltpu.VMEM((CB, NCH, CHUNK), jnp.int32),
            pltpu.VMEM((CB, NCH, 2 * CHUNK), jnp.int32),
            [pltpu.VMEM((2 * CHUNK, WR), jnp.float32)] * NBUF
            + [pltpu.SemaphoreType.DMA] * (1 + 2 * NBUF),
        ],
        compiler_params=pltpu.CompilerParams(needs_layout_passes=False),
    )(item, cl0, cl1, cl2, w0, w1, w2)


def kernel(item, code_list_0, code_list_1, code_list_2, W0, W1, W2):
    raw = _rpq(item, code_list_0, code_list_1, code_list_2,
               W0.reshape(MCW, WR), W1.reshape(MCW, WR), W2.reshape(MCW, WR))
    return jnp.moveaxis(raw.reshape(CB, B, D), 0, -1)


# speedup vs baseline: 1.8161x; 1.6517x over previous
import functools

import jax
import jax.numpy as jnp
from jax import lax
from jax.experimental import pallas as pl
from jax.experimental.pallas import tpu as pltpu
from jax.experimental.pallas import tpu_sc as plsc

B = 16384
D = 256
CB = 3
NC = 2
NS = 16
NW = NC * NS
BPW = B // NW
CHUNK = 64
NCH = BPW // CHUNK
LANES = 16
NBUF = 7
LOOKAHEAD = 4
WR = 128
MCW = 8192 * D // WR
NSTEP = NCH * CB


def _body(item_hbm, cl0, cl1, cl2, w0, w1, w2, out_hbm,
          item_v, codes_v, idx2_v, embs_and_sems):
    embufs = embs_and_sems[:NBUF]
    sem_c = embs_and_sems[NBUF]
    gsems = embs_and_sems[NBUF + 1:2 * NBUF + 1]
    osems = embs_and_sems[2 * NBUF + 1:]
    wid = lax.axis_index("s") * NC + lax.axis_index("c")
    cls = (cl0, cl1, cl2)
    ws = (w0, w1, w2)
    evens = 2 * lax.iota(jnp.int32, LANES)

    pltpu.async_copy(item_hbm.at[pl.ds(wid * BPW, BPW)], item_v, sem_c).wait()

    cdescs = []
    for c in range(NCH):
        for i in range(CB):
            cdescs.append(pltpu.async_copy(
                cls[i].at[item_v.at[pl.ds(c * CHUNK, CHUNK)]],
                codes_v.at[i, c], sem_c))
    for d in cdescs:
        d.wait()

    iota16 = lax.iota(jnp.int32, LANES)
    p8 = lax.rem(iota16, 8)
    tv = lax.div(iota16, 8)
    for i in range(CB):
        iv = jnp.full((LANES,), i, jnp.int32)
        for c in range(NCH):
            cv_ = jnp.full((LANES,), c, jnp.int32)
            for a in range(CHUNK // 8):
                cv = plsc.load_gather(codes_v, [iv, cv_, a * 8 + p8])
                plsc.store_scatter(idx2_v.at[i, c], [iota16 + a * LANES],
                                   cv + cv + tv)

    steps = [(c, i) for c in range(NCH) for i in range(CB)]

    def fire(s):
        c, i = steps[s]
        b = s % NBUF
        return pltpu.async_copy(ws[i].at[idx2_v.at[i, c]], embufs[b], gsems[b])

    def fire_out(s):
        c, i = steps[s]
        b = s % NBUF
        row0 = i * 2 * B + 2 * (wid * BPW + c * CHUNK)
        return pltpu.async_copy(
            embufs[b], out_hbm.at[pl.ds(row0, 2 * CHUNK)], osems[b])

    descs = {}
    odescs = {}
    for s in range(LOOKAHEAD):
        descs[s] = fire(s)
    for s in range(NSTEP):
        descs[s].wait()
        odescs[s] = fire_out(s)
        t = s + LOOKAHEAD
        if t < NSTEP:
            if t >= NBUF:
                odescs[t - NBUF].wait()
            descs[t] = fire(t)
    for s in range(NSTEP - NBUF, NSTEP):
        odescs[s].wait()


@jax.jit
def _rpq(item, cl0, cl1, cl2, w0, w1, w2):
    mesh = plsc.VectorSubcoreMesh(
        core_axis_name="c", subcore_axis_name="s",
        num_cores=NC, num_subcores=NS)
    return pl.kernel(
        _body,
        out_type=jax.ShapeDtypeStruct((CB * 2 * B, WR), jnp.float32),
        mesh=mesh,
        scratch_types=[
            pltpu.VMEM((BPW,), jnp.int32),
            pltpu.VMEM((CB, NCH, CHUNK), jnp.int32),
            pltpu.VMEM((CB, NCH, 2 * CHUNK), jnp.int32),
            [pltpu.VMEM((2 * CHUNK, WR), jnp.float32)] * NBUF
            + [pltpu.SemaphoreType.DMA] * (1 + 2 * NBUF),
        ],
        compiler_params=pltpu.CompilerParams(needs_layout_passes=False),
    )(item, cl0, cl1, cl2, w0, w1, w2)


def kernel(item, code_list_0, code_list_1, code_list_2, W0, W1, W2):
    raw = _rpq(item, code_list_0, code_list_1, code_list_2,
               W0.reshape(MCW, WR), W1.reshape(MCW, WR), W2.reshape(MCW, WR))
    out = raw.reshape(CB, B // 8, 2, 8, WR).transpose(0, 1, 3, 2, 4)
    return jnp.moveaxis(out.reshape(CB, B, D), 0, -1)


# speedup vs baseline: 2.5034x; 1.3784x over previous
import functools

import jax
import jax.numpy as jnp
from jax import lax
from jax.experimental import pallas as pl
from jax.experimental.pallas import tpu as pltpu
from jax.experimental.pallas import tpu_sc as plsc

B = 16384
D = 256
CB = 3
NC = 2
NS = 16
NW = NC * NS
BPW = B // NW
CHUNK = 64
NCH = BPW // CHUNK
LANES = 16
NBUF = 7
LOOKAHEAD = 4
WR = 128
MCW = 8192 * D // WR
NSTEP = NCH * CB


def _body(item_hbm, cl0, cl1, cl2, w0, w1, w2, out_hbm,
          item_v, codes_v, idx2_v, embs_and_sems):
    embufs = embs_and_sems[:NBUF]
    sem_c = embs_and_sems[NBUF]
    gsems = embs_and_sems[NBUF + 1:2 * NBUF + 1]
    osems = embs_and_sems[2 * NBUF + 1:]
    wid = lax.axis_index("s") * NC + lax.axis_index("c")
    cls = (cl0, cl1, cl2)
    ws = (w0, w1, w2)
    evens = 2 * lax.iota(jnp.int32, LANES)

    pltpu.async_copy(item_hbm.at[pl.ds(wid * BPW, BPW)], item_v, sem_c).wait()

    cdescs = []
    for c in range(NCH):
        for i in range(CB):
            cdescs.append(pltpu.async_copy(
                cls[i].at[item_v.at[pl.ds(c * CHUNK, CHUNK)]],
                codes_v.at[i, c], sem_c))
    for d in cdescs:
        d.wait()

    iota16 = lax.iota(jnp.int32, LANES)
    p8 = lax.rem(iota16, 8)
    tv = lax.div(iota16, 8)
    for i in range(CB):
        iv = jnp.full((LANES,), i, jnp.int32)
        for c in range(NCH):
            cv_ = jnp.full((LANES,), c, jnp.int32)
            for a in range(CHUNK // 8):
                cv = plsc.load_gather(codes_v, [iv, cv_, a * 8 + p8])
                plsc.store_scatter(idx2_v.at[i, c], [iota16 + a * LANES],
                                   cv + (cv & (-8)) + 8 * tv)

    steps = [(c, i) for c in range(NCH) for i in range(CB)]

    def fire(s):
        c, i = steps[s]
        b = s % NBUF
        return pltpu.async_copy(ws[i].at[idx2_v.at[i, c]], embufs[b], gsems[b])

    def fire_out(s):
        c, i = steps[s]
        b = s % NBUF
        row0 = i * 2 * B + 2 * (wid * BPW + c * CHUNK)
        return pltpu.async_copy(
            embufs[b], out_hbm.at[pl.ds(row0, 2 * CHUNK)], osems[b])

    descs = {}
    odescs = {}
    for s in range(LOOKAHEAD):
        descs[s] = fire(s)
    for s in range(NSTEP):
        descs[s].wait()
        odescs[s] = fire_out(s)
        t = s + LOOKAHEAD
        if t < NSTEP:
            if t >= NBUF:
                odescs[t - NBUF].wait()
            descs[t] = fire(t)
    for s in range(NSTEP - NBUF, NSTEP):
        odescs[s].wait()


@jax.jit
def _rpq(item, cl0, cl1, cl2, w0, w1, w2):
    mesh = plsc.VectorSubcoreMesh(
        core_axis_name="c", subcore_axis_name="s",
        num_cores=NC, num_subcores=NS)
    return pl.kernel(
        _body,
        out_type=jax.ShapeDtypeStruct((CB * 2 * B, WR), jnp.float32),
        mesh=mesh,
        scratch_types=[
            pltpu.VMEM((BPW,), jnp.int32),
            pltpu.VMEM((CB, NCH, CHUNK), jnp.int32),
            pltpu.VMEM((CB, NCH, 2 * CHUNK), jnp.int32),
            [pltpu.VMEM((2 * CHUNK, WR), jnp.float32)] * NBUF
            + [pltpu.SemaphoreType.DMA] * (1 + 2 * NBUF),
        ],
        compiler_params=pltpu.CompilerParams(needs_layout_passes=False),
    )(item, cl0, cl1, cl2, w0, w1, w2)


def kernel(item, code_list_0, code_list_1, code_list_2, W0, W1, W2):
    def wview(W):
        return W.reshape(8192 // 8, 8, 2, WR).transpose(0, 2, 1, 3).reshape(MCW, WR)

    raw = _rpq(item, code_list_0, code_list_1, code_list_2,
               wview(W0), wview(W1), wview(W2))
    out = raw.reshape(CB, B // 8, 2, 8, WR).transpose(0, 1, 3, 2, 4)
    return jnp.moveaxis(out.reshape(CB, B, D), 0, -1)
